# baseline (speedup 1.0000x reference)
import jax
import jax.numpy as jnp
from jax import lax
from jax.experimental import pallas as pl
from jax.experimental.pallas import tpu as pltpu
from jax.experimental.pallas import tpu_sc as plsc

VOCAB = 100000
DIM = 64
B = 1024
L = 50

NC = 2
NS = 16
NW = NC * NS
B_PER_W = B // NW
ROWS_PER_W = B_PER_W * L
CHUNK = 80
NCHUNK = ROWS_PER_W // CHUNK


def _bag_body(x_hbm, embed_hbm, out_hbm, idx_v, rows_v, acc_v, sem):
    wid = lax.axis_index("s") * NC + lax.axis_index("c")
    base = wid * ROWS_PER_W

    pltpu.sync_copy(x_hbm.at[pl.ds(base, ROWS_PER_W)], idx_v)

    copies = []
    for k in range(NCHUNK):
        src = embed_hbm.at[idx_v.at[pl.ds(k * CHUNK, CHUNK)]]
        dst = rows_v.at[pl.ds(k * CHUNK, CHUNK)]
        copies.append(pltpu.async_copy(src, dst, sem))
    for c in copies:
        c.wait()

    def body(bi, _):
        r0 = bi * L
        accs = [rows_v[r0, pl.ds(c * 16, 16)] for c in range(DIM // 16)]
        for l in range(1, L):
            for c in range(DIM // 16):
                accs[c] = accs[c] + rows_v[r0 + l, pl.ds(c * 16, 16)]
        for c in range(DIM // 16):
            acc_v[bi, pl.ds(c * 16, 16)] = accs[c]
        return 0

    lax.fori_loop(0, B_PER_W, body, 0)

    pltpu.sync_copy(acc_v, out_hbm.at[pl.ds(wid * B_PER_W, B_PER_W)])


@jax.jit
def _bag(x_flat, embed):
    mesh = plsc.VectorSubcoreMesh(
        core_axis_name="c", subcore_axis_name="s", num_cores=NC, num_subcores=NS
    )
    return pl.kernel(
        _bag_body,
        out_type=jax.ShapeDtypeStruct((B, DIM), jnp.float32),
        mesh=mesh,
        scratch_types=[
            pltpu.VMEM((ROWS_PER_W,), jnp.int32),
            pltpu.VMEM((ROWS_PER_W, DIM), jnp.float32),
            pltpu.VMEM((B_PER_W, DIM), jnp.float32),
            pltpu.SemaphoreType.DMA,
        ],
        compiler_params=pltpu.CompilerParams(use_tc_tiling_on_sc=False),
    )(x_flat, embed)


TB = 2048


def _lin_body(et_ref, o_ref):
    t = lax.dot_general(
        et_ref[...],
        jnp.eye(DIM, dtype=jnp.float32),
        (((0,), (0,)), ((), ())),
        preferred_element_type=jnp.float32,
    )
    o_ref[...] = jnp.concatenate([t, jnp.zeros((TB, DIM), jnp.float32)], axis=1)


@jax.jit
def _linearize(et):
    nb = pl.cdiv(VOCAB, TB)
    return pl.pallas_call(
        _lin_body,
        grid=(nb,),
        in_specs=[pl.BlockSpec((DIM, TB), lambda i: (0, i))],
        out_specs=pl.BlockSpec((TB, 2 * DIM), lambda i: (i, 0)),
        out_shape=jax.ShapeDtypeStruct((VOCAB, 2 * DIM), jnp.float32),
    )(et)


BV = 4096


def _mm_body(wt_ref, s_ref, b_ref, o_ref):
    o_ref[...] = lax.dot_general(
        wt_ref[...],
        s_ref[...],
        (((0,), (1,)), ((), ())),
        preferred_element_type=jnp.float32,
    ) + lax.broadcast_in_dim(b_ref[...], (BV, B), (0,))


@jax.jit
def _project(s, Wt, b):
    nv = pl.cdiv(VOCAB, BV)
    out_t = pl.pallas_call(
        _mm_body,
        grid=(nv,),
        in_specs=[
            pl.BlockSpec((DIM, BV), lambda i: (0, i)),
            pl.BlockSpec((B, DIM), lambda i: (0, 0)),
            pl.BlockSpec((BV,), lambda i: (i,)),
        ],
        out_specs=pl.BlockSpec((BV, B), lambda i: (i, 0)),
        out_shape=jax.ShapeDtypeStruct((VOCAB, B), jnp.float32),
    )(Wt, s, b)
    return out_t.T


def kernel(x, embed, W, b):
    x_flat = 2 * x.reshape(-1).astype(jnp.int32)
    embed_lin = _linearize(embed.T).reshape(2 * VOCAB, DIM)
    s = _bag(x_flat, embed_lin)
    return _project(s, W.T, b)


# speedup vs baseline: 1.1111x; 1.1111x over previous
import jax
import jax.numpy as jnp
from jax import lax
from jax.experimental import pallas as pl
from jax.experimental.pallas import tpu as pltpu
from jax.experimental.pallas import tpu_sc as plsc

VOCAB = 100000
DIM = 64
B = 1024
L = 50

NC = 2
NS = 16
NW = NC * NS
B_PER_W = B // NW
ROWS_PER_W = B_PER_W * L
CHUNK = 80
NCHUNK = ROWS_PER_W // CHUNK


def _bag_body(x_hbm, embed_hbm, out_hbm, idx_v, rows_v, acc_v, sem):
    wid = lax.axis_index("s") * NC + lax.axis_index("c")
    base = wid * ROWS_PER_W

    pltpu.sync_copy(x_hbm.at[pl.ds(base, ROWS_PER_W)], idx_v)

    copies = []
    for k in range(NCHUNK):
        src = embed_hbm.at[idx_v.at[pl.ds(k * CHUNK, CHUNK)]]
        dst = rows_v.at[pl.ds(k * CHUNK, CHUNK)]
        copies.append(pltpu.async_copy(src, dst, sem))
    for c in copies:
        c.wait()

    def body(bi, _):
        r0 = bi * L
        accs = [rows_v[r0, pl.ds(c * 16, 16)] for c in range(DIM // 16)]
        for l in range(1, L):
            for c in range(DIM // 16):
                accs[c] = accs[c] + rows_v[r0 + l, pl.ds(c * 16, 16)]
        for c in range(DIM // 16):
            acc_v[bi, pl.ds(c * 16, 16)] = accs[c]
        return 0

    lax.fori_loop(0, B_PER_W, body, 0)

    pltpu.sync_copy(acc_v, out_hbm.at[pl.ds(wid * B_PER_W, B_PER_W)])


@jax.jit
def _bag(x_flat, embed):
    mesh = plsc.VectorSubcoreMesh(
        core_axis_name="c", subcore_axis_name="s", num_cores=NC, num_subcores=NS
    )
    return pl.kernel(
        _bag_body,
        out_type=jax.ShapeDtypeStruct((B, DIM), jnp.float32),
        mesh=mesh,
        scratch_types=[
            pltpu.VMEM((ROWS_PER_W,), jnp.int32),
            pltpu.VMEM((ROWS_PER_W, DIM), jnp.float32),
            pltpu.VMEM((B_PER_W, DIM), jnp.float32),
            pltpu.SemaphoreType.DMA,
        ],
        compiler_params=pltpu.CompilerParams(use_tc_tiling_on_sc=False),
    )(x_flat, embed)


TB = 8192


def _lin_body(et_ref, o_ref):
    t = jnp.transpose(et_ref[...])
    o_ref[...] = jnp.concatenate([t, jnp.zeros((TB, DIM), jnp.float32)], axis=1)


@jax.jit
def _linearize(et):
    nb = pl.cdiv(VOCAB, TB)
    return pl.pallas_call(
        _lin_body,
        grid=(nb,),
        in_specs=[pl.BlockSpec((DIM, TB), lambda i: (0, i))],
        out_specs=pl.BlockSpec((TB, 2 * DIM), lambda i: (i, 0)),
        out_shape=jax.ShapeDtypeStruct((VOCAB, 2 * DIM), jnp.float32),
    )(et)


BV = 4096


def _mm_body(wt_ref, s_ref, b_ref, o_ref):
    o_ref[...] = lax.dot_general(
        wt_ref[...],
        s_ref[...],
        (((0,), (1,)), ((), ())),
        preferred_element_type=jnp.float32,
    ) + lax.broadcast_in_dim(b_ref[...], (BV, B), (0,))


@jax.jit
def _project(s, Wt, b):
    nv = pl.cdiv(VOCAB, BV)
    out_t = pl.pallas_call(
        _mm_body,
        grid=(nv,),
        in_specs=[
            pl.BlockSpec((DIM, BV), lambda i: (0, i)),
            pl.BlockSpec((B, DIM), lambda i: (0, 0)),
            pl.BlockSpec((BV,), lambda i: (i,)),
        ],
        out_specs=pl.BlockSpec((BV, B), lambda i: (i, 0)),
        out_shape=jax.ShapeDtypeStruct((VOCAB, B), jnp.float32),
    )(Wt, s, b)
    return out_t.T


def kernel(x, embed, W, b):
    x_flat = 2 * x.reshape(-1).astype(jnp.int32)
    embed_lin = _linearize(embed.T).reshape(2 * VOCAB, DIM)
    s = _bag(x_flat, embed_lin)
    return _project(s, W.T, b)


# speedup vs baseline: 1.1170x; 1.0054x over previous
import jax
import jax.numpy as jnp
from jax import lax
from jax.experimental import pallas as pl
from jax.experimental.pallas import tpu as pltpu
from jax.experimental.pallas import tpu_sc as plsc

VOCAB = 100000
DIM = 64
B = 1024
L = 50

NC = 2
NS = 16
NW = NC * NS
B_PER_W = B // NW
ROWS_PER_W = B_PER_W * L
CHUNK = 80
NCHUNK = ROWS_PER_W // CHUNK


def _bag_body(x_hbm, embed_hbm, out_hbm, idx_v, rows_v, acc_v, sem):
    wid = lax.axis_index("s") * NC + lax.axis_index("c")
    base = wid * ROWS_PER_W

    pltpu.sync_copy(x_hbm.at[pl.ds(base, ROWS_PER_W)], idx_v)

    copies = []
    for k in range(NCHUNK):
        src = embed_hbm.at[idx_v.at[pl.ds(k * CHUNK, CHUNK)]]
        dst = rows_v.at[pl.ds(k * CHUNK, CHUNK)]
        copies.append(pltpu.async_copy(src, dst, sem))
    for c in copies:
        c.wait()

    def body(bi, _):
        r0 = bi * L
        accs = [rows_v[r0, pl.ds(c * 16, 16)] for c in range(DIM // 16)]
        for l in range(1, L):
            for c in range(DIM // 16):
                accs[c] = accs[c] + rows_v[r0 + l, pl.ds(c * 16, 16)]
        for c in range(DIM // 16):
            acc_v[bi, pl.ds(c * 16, 16)] = accs[c]
        return 0

    lax.fori_loop(0, B_PER_W, body, 0)

    pltpu.sync_copy(acc_v, out_hbm.at[pl.ds(wid * B_PER_W, B_PER_W)])


@jax.jit
def _bag(x_flat, embed):
    mesh = plsc.VectorSubcoreMesh(
        core_axis_name="c", subcore_axis_name="s", num_cores=NC, num_subcores=NS
    )
    return pl.kernel(
        _bag_body,
        out_type=jax.ShapeDtypeStruct((B, DIM), jnp.float32),
        mesh=mesh,
        scratch_types=[
            pltpu.VMEM((ROWS_PER_W,), jnp.int32),
            pltpu.VMEM((ROWS_PER_W, DIM), jnp.float32),
            pltpu.VMEM((B_PER_W, DIM), jnp.float32),
            pltpu.SemaphoreType.DMA,
        ],
        compiler_params=pltpu.CompilerParams(use_tc_tiling_on_sc=False),
    )(x_flat, embed)


TB = 16384


def _lin_body(et_ref, o_ref):
    t = jnp.transpose(et_ref[...])
    o_ref[...] = jnp.concatenate([t, jnp.zeros((TB, DIM), jnp.float32)], axis=1)


@jax.jit
def _linearize(et):
    nb = pl.cdiv(VOCAB, TB)
    return pl.pallas_call(
        _lin_body,
        grid=(nb,),
        in_specs=[pl.BlockSpec((DIM, TB), lambda i: (0, i))],
        out_specs=pl.BlockSpec((TB, 2 * DIM), lambda i: (i, 0)),
        out_shape=jax.ShapeDtypeStruct((VOCAB, 2 * DIM), jnp.float32),
    )(et)


BV = 4096


def _mm_body(wt_ref, s_ref, b_ref, o_ref):
    o_ref[...] = lax.dot_general(
        wt_ref[...],
        s_ref[...],
        (((0,), (1,)), ((), ())),
        preferred_element_type=jnp.float32,
    ) + lax.broadcast_in_dim(b_ref[...], (BV, B), (0,))


@jax.jit
def _project(s, Wt, b):
    nv = pl.cdiv(VOCAB, BV)
    out_t = pl.pallas_call(
        _mm_body,
        grid=(nv,),
        in_specs=[
            pl.BlockSpec((DIM, BV), lambda i: (0, i)),
            pl.BlockSpec((B, DIM), lambda i: (0, 0)),
            pl.BlockSpec((BV,), lambda i: (i,)),
        ],
        out_specs=pl.BlockSpec((BV, B), lambda i: (i, 0)),
        out_shape=jax.ShapeDtypeStruct((VOCAB, B), jnp.float32),
    )(Wt, s, b)
    return out_t.T


def kernel(x, embed, W, b):
    x_flat = 2 * x.reshape(-1).astype(jnp.int32)
    embed_lin = _linearize(embed.T).reshape(2 * VOCAB, DIM)
    s = _bag(x_flat, embed_lin)
    return _project(s, W.T, b)


# speedup vs baseline: 1.1249x; 1.0071x over previous
import jax
import jax.numpy as jnp
from jax import lax
from jax.experimental import pallas as pl
from jax.experimental.pallas import tpu as pltpu
from jax.experimental.pallas import tpu_sc as plsc

VOCAB = 100000
DIM = 64
B = 1024
L = 50

NC = 2
NS = 16
NW = NC * NS
B_PER_W = B // NW
ROWS_PER_W = B_PER_W * L
CHUNK = 80
NCHUNK = ROWS_PER_W // CHUNK


def _bag_body(x_hbm, embed_hbm, out_hbm, idx_v, rows_v, acc_v, sem):
    wid = lax.axis_index("s") * NC + lax.axis_index("c")
    base = wid * ROWS_PER_W

    pltpu.sync_copy(x_hbm.at[pl.ds(base, ROWS_PER_W)], idx_v)

    copies = []
    for k in range(NCHUNK):
        src = embed_hbm.at[idx_v.at[pl.ds(k * CHUNK, CHUNK)]]
        dst = rows_v.at[pl.ds(k * CHUNK, CHUNK)]
        copies.append(pltpu.async_copy(src, dst, sem))

    def body(bi, _):
        r0 = bi * L
        accs = [rows_v[r0, pl.ds(c * 16, 16)] for c in range(DIM // 16)]
        for l in range(1, L):
            for c in range(DIM // 16):
                accs[c] = accs[c] + rows_v[r0 + l, pl.ds(c * 16, 16)]
        for c in range(DIM // 16):
            acc_v[bi, pl.ds(c * 16, 16)] = accs[c]
        return 0

    half = NCHUNK // 2
    for c in copies[:half]:
        c.wait()
    lax.fori_loop(0, (half * CHUNK) // L, body, 0)
    for c in copies[half:]:
        c.wait()
    lax.fori_loop((half * CHUNK) // L, B_PER_W, body, 0)

    pltpu.sync_copy(acc_v, out_hbm.at[pl.ds(wid * B_PER_W, B_PER_W)])


@jax.jit
def _bag(x_flat, embed):
    mesh = plsc.VectorSubcoreMesh(
        core_axis_name="c", subcore_axis_name="s", num_cores=NC, num_subcores=NS
    )
    return pl.kernel(
        _bag_body,
        out_type=jax.ShapeDtypeStruct((B, DIM), jnp.float32),
        mesh=mesh,
        scratch_types=[
            pltpu.VMEM((ROWS_PER_W,), jnp.int32),
            pltpu.VMEM((ROWS_PER_W, DIM), jnp.float32),
            pltpu.VMEM((B_PER_W, DIM), jnp.float32),
            pltpu.SemaphoreType.DMA,
        ],
        compiler_params=pltpu.CompilerParams(use_tc_tiling_on_sc=False),
    )(x_flat, embed)


TB = 16384


def _lin_body(et_ref, o_ref):
    t = jnp.transpose(et_ref[...])
    o_ref[...] = jnp.concatenate([t, jnp.zeros((TB, DIM), jnp.float32)], axis=1)


@jax.jit
def _linearize(et):
    nb = pl.cdiv(VOCAB, TB)
    return pl.pallas_call(
        _lin_body,
        grid=(nb,),
        in_specs=[pl.BlockSpec((DIM, TB), lambda i: (0, i))],
        out_specs=pl.BlockSpec((TB, 2 * DIM), lambda i: (i, 0)),
        out_shape=jax.ShapeDtypeStruct((VOCAB, 2 * DIM), jnp.float32),
    )(et)


BV = 4096


def _mm_body(wt_ref, s_ref, b_ref, o_ref):
    o_ref[...] = lax.dot_general(
        wt_ref[...],
        s_ref[...],
        (((0,), (1,)), ((), ())),
        preferred_element_type=jnp.float32,
    ) + lax.broadcast_in_dim(b_ref[...], (BV, B), (0,))


@jax.jit
def _project(s, Wt, b):
    nv = pl.cdiv(VOCAB, BV)
    out_t = pl.pallas_call(
        _mm_body,
        grid=(nv,),
        in_specs=[
            pl.BlockSpec((DIM, BV), lambda i: (0, i)),
            pl.BlockSpec((B, DIM), lambda i: (0, 0)),
            pl.BlockSpec((BV,), lambda i: (i,)),
        ],
        out_specs=pl.BlockSpec((BV, B), lambda i: (i, 0)),
        out_shape=jax.ShapeDtypeStruct((VOCAB, B), jnp.float32),
    )(Wt, s, b)
    return out_t.T


def kernel(x, embed, W, b):
    x_flat = 2 * x.reshape(-1).astype(jnp.int32)
    embed_lin = _linearize(embed.T).reshape(2 * VOCAB, DIM)
    s = _bag(x_flat, embed_lin)
    return _project(s, W.T, b)
